# baseline (speedup 1.0000x reference)
import functools

import jax
import jax.numpy as jnp
from jax import lax
from jax.experimental import pallas as pl
from jax.experimental.pallas import tpu as pltpu
from jax.experimental.pallas import tpu_sc as plsc

_N = 10000
_E = 320000
_D = 128

_NC = 2
_NS = 16
_NW = _NC * _NS
_CH = 128
_C = 80
_T = 16
_THIEF = 1
_EP = _NW * _C * _CH
_DUMMY = _N

_ACC_ROWS = 10240
_OUT_PER_TILE = _ACC_ROWS // _NS

_DEG_PER_TILE = 632
_DEGP = _NS * _DEG_PER_TILE


def _sc_degree_impl(dst_hbm, deg_hbm, dst_v, zeros_v, ones_v, acc):
    cid = lax.axis_index("c")
    sid = lax.axis_index("s")

    def _fill(i, _):
        zeros_v[pl.ds(i * 16, 16)] = jnp.zeros((16,), jnp.float32)
        return 0
    lax.fori_loop(0, 640 // 16, _fill, 0)

    def _fill1(i, _):
        ones_v[pl.ds(i * 16, 16)] = jnp.ones((16,), jnp.float32)
        return 0
    lax.fori_loop(0, _CH // 16, _fill1, 0)

    pltpu.sync_copy(zeros_v.at[pl.ds(0, _DEG_PER_TILE)],
                    acc.at[pl.ds(sid * _DEG_PER_TILE, _DEG_PER_TILE)])
    plsc.subcore_barrier()

    wid = sid * _NC + cid
    pltpu.sync_copy(dst_hbm.at[wid], dst_v)

    def _body(j, _):
        pltpu.sync_copy(ones_v, acc.at[dst_v.at[j]], add=True)
        return 0
    lax.fori_loop(0, _C, _body, 0)
    plsc.subcore_barrier()

    pltpu.sync_copy(acc.at[pl.ds(sid * _DEG_PER_TILE, _DEG_PER_TILE)],
                    zeros_v.at[pl.ds(0, _DEG_PER_TILE)])
    pltpu.sync_copy(zeros_v.at[pl.ds(0, _DEG_PER_TILE)],
                    deg_hbm.at[pl.ds(cid * _DEGP + sid * _DEG_PER_TILE,
                                     _DEG_PER_TILE)])


def _sc_neighbor_sum_impl(hs_hbm, src_hbm, dst_hbm, out_hbm,
                          src_v, dst_v, buf, acc, sem):
    cid = lax.axis_index("c")
    sid = lax.axis_index("s")

    def _fill(i, _):
        r = i // 8
        c = lax.rem(i, 8) * 16
        buf[r, pl.ds(c, 16)] = jnp.zeros((16,), jnp.float32)
        return 0
    lax.fori_loop(0, _CH * 8, _fill, 0)

    zb = sid * _OUT_PER_TILE
    for t in range(_OUT_PER_TILE // _CH):
        pltpu.sync_copy(buf, acc.at[pl.ds(zb + t * _CH, _CH)])
    plsc.subcore_barrier()

    wid = sid * _NC + cid
    pwid = sid * _NC + (1 - _THIEF)
    pltpu.sync_copy(src_hbm.at[wid], src_v.at[pl.ds(0, _C)])
    pltpu.sync_copy(dst_hbm.at[wid], dst_v.at[pl.ds(0, _C)])

    @pl.when(cid == _THIEF)
    def _():
        pltpu.sync_copy(src_hbm.at[pwid, pl.ds(_C - _T, _T)],
                        src_v.at[pl.ds(_C, _T)])
        pltpu.sync_copy(dst_hbm.at[pwid, pl.ds(_C - _T, _T)],
                        dst_v.at[pl.ds(_C, _T)])

    def _body(j, _):
        pltpu.async_copy(hs_hbm.at[src_v.at[j]], buf, sem).wait()
        pltpu.sync_copy(buf, acc.at[dst_v.at[j]], add=True)
        return 0

    @pl.when(cid == _THIEF)
    def _():
        lax.fori_loop(0, _C + _T, _body, 0)

    @pl.when(cid != _THIEF)
    def _():
        lax.fori_loop(0, _C - _T, _body, 0)
    plsc.subcore_barrier()

    ob = sid * _OUT_PER_TILE
    for t in range(_OUT_PER_TILE // _CH):
        pltpu.sync_copy(acc.at[pl.ds(ob + t * _CH, _CH)], buf)
        pltpu.sync_copy(buf, out_hbm.at[cid, pl.ds(ob + t * _CH, _CH)])


_BLK = 1000
_GRID = _N // _BLK


def _tc_first_body(d0_ref, d1_ref, x_ref, w_ref, hs_ref, dinv_ref):
    deg = d0_ref[...] + d1_ref[...] + 1.0
    dinv = lax.rsqrt(jnp.maximum(deg, 1.0))
    h = jnp.dot(x_ref[...], w_ref[...], preferred_element_type=jnp.float32)
    hs_ref[...] = dinv * h
    dinv_ref[...] = dinv


def _tc_mid_body(p0_ref, p1_ref, hs_ref, dinv_ref, b_ref, w_ref, out_ref):
    dinv = dinv_ref[...]
    agg = p0_ref[...] + p1_ref[...] + hs_ref[...]
    t = jnp.maximum(dinv * agg + b_ref[...], 0.0)
    out_ref[...] = dinv * jnp.dot(t, w_ref[...], preferred_element_type=jnp.float32)


def _tc_final_body(p0_ref, p1_ref, hs_ref, dinv_ref, b_ref, out_ref):
    agg = p0_ref[...] + p1_ref[...] + hs_ref[...]
    out_ref[...] = dinv_ref[...] * agg + b_ref[...]


_row_spec = pl.BlockSpec((_BLK, _D), lambda i: (i, 0))
_col_spec = pl.BlockSpec((_BLK, 1), lambda i: (i, 0))
_w_spec = pl.BlockSpec((_D, _D), lambda i: (0, 0))
_b_spec = pl.BlockSpec((1, _D), lambda i: (0, 0))

_tc_first = pl.pallas_call(
    _tc_first_body,
    grid=(_GRID,),
    in_specs=[_col_spec, _col_spec, _row_spec, _w_spec],
    out_specs=[_row_spec, _col_spec],
    out_shape=[jax.ShapeDtypeStruct((_N, _D), jnp.float32),
               jax.ShapeDtypeStruct((_N, 1), jnp.float32)],
)

_tc_mid = pl.pallas_call(
    _tc_mid_body,
    grid=(_GRID,),
    in_specs=[_row_spec, _row_spec, _row_spec, _col_spec, _b_spec, _w_spec],
    out_specs=_row_spec,
    out_shape=jax.ShapeDtypeStruct((_N, _D), jnp.float32),
)

_tc_final = pl.pallas_call(
    _tc_final_body,
    grid=(_GRID,),
    in_specs=[_row_spec, _row_spec, _row_spec, _col_spec, _b_spec],
    out_specs=_row_spec,
    out_shape=jax.ShapeDtypeStruct((_N, _D), jnp.float32),
)


@functools.lru_cache(maxsize=1)
def _sc_kernels():
    mesh = plsc.VectorSubcoreMesh(
        core_axis_name="c", subcore_axis_name="s",
        num_cores=_NC, num_subcores=_NS)
    sc_degree = pl.kernel(
        _sc_degree_impl,
        out_type=jax.ShapeDtypeStruct((_NC * _DEGP,), jnp.float32),
        mesh=mesh,
        scratch_types=[
            pltpu.VMEM((_C, _CH), jnp.int32),
            pltpu.VMEM((640,), jnp.float32),
            pltpu.VMEM((_CH,), jnp.float32),
            pltpu.VMEM_SHARED((_DEGP,), jnp.float32),
        ],
    )
    sc_neighbor_sum = pl.kernel(
        _sc_neighbor_sum_impl,
        out_type=jax.ShapeDtypeStruct((_NC, _ACC_ROWS, _D), jnp.float32),
        mesh=mesh,
        scratch_types=[
            pltpu.VMEM((_C + _T, _CH), jnp.int32),
            pltpu.VMEM((_C + _T, _CH), jnp.int32),
            pltpu.VMEM((_CH, _D), jnp.float32),
            pltpu.VMEM_SHARED((_ACC_ROWS, _D), jnp.float32),
            pltpu.SemaphoreType.DMA,
        ],
    )
    return sc_degree, sc_neighbor_sum


def kernel(x, edge_index, W1, b1, W2, b2, W3, b3):
    src = edge_index[0]
    dst = edge_index[1]
    pad = _EP - _E
    srcp = jnp.concatenate(
        [src, jnp.zeros((pad,), jnp.int32)]).reshape(_NW, _C, _CH)
    dstp = jnp.concatenate(
        [dst, jnp.full((pad,), _DUMMY, jnp.int32)]).reshape(_NW, _C, _CH)

    _sc_degree, _sc_neighbor_sum = _sc_kernels()
    deg2 = _sc_degree(dstp).reshape(_NC, _DEGP)
    d0 = deg2[0, :_N, None]
    d1 = deg2[1, :_N, None]

    b1r = b1.reshape(1, _D)
    b2r = b2.reshape(1, _D)
    b3r = b3.reshape(1, _D)

    hs1, dinv = _tc_first(d0, d1, x, W1)
    parts = _sc_neighbor_sum(hs1, srcp, dstp)
    hs2 = _tc_mid(parts[0, :_N], parts[1, :_N], hs1, dinv, b1r, W2)
    parts = _sc_neighbor_sum(hs2, srcp, dstp)
    hs3 = _tc_mid(parts[0, :_N], parts[1, :_N], hs2, dinv, b2r, W3)
    parts = _sc_neighbor_sum(hs3, srcp, dstp)
    return _tc_final(parts[0, :_N], parts[1, :_N], hs3, dinv, b3r)


# speedup vs baseline: 1.0513x; 1.0513x over previous
import functools

import jax
import jax.numpy as jnp
from jax import lax
from jax.experimental import pallas as pl
from jax.experimental.pallas import tpu as pltpu
from jax.experimental.pallas import tpu_sc as plsc

_N = 10000
_E = 320000
_D = 128

_NC = 2
_NS = 16
_NW = _NC * _NS
_CH = 128
_C = 80
_EP = _NW * _C * _CH
_DUMMY = _N

_ACC_ROWS = 10240
_OUT_PER_TILE = _ACC_ROWS // _NS

_DEG_PER_TILE = 632
_DEGP = _NS * _DEG_PER_TILE


def _sc_degree_impl(dst_hbm, deg_hbm, dst_v, zeros_v, ones_v, acc):
    cid = lax.axis_index("c")
    sid = lax.axis_index("s")

    def _fill(i, _):
        zeros_v[pl.ds(i * 16, 16)] = jnp.zeros((16,), jnp.float32)
        return 0
    lax.fori_loop(0, 640 // 16, _fill, 0)

    def _fill1(i, _):
        ones_v[pl.ds(i * 16, 16)] = jnp.ones((16,), jnp.float32)
        return 0
    lax.fori_loop(0, _CH // 16, _fill1, 0)

    pltpu.sync_copy(zeros_v.at[pl.ds(0, _DEG_PER_TILE)],
                    acc.at[pl.ds(sid * _DEG_PER_TILE, _DEG_PER_TILE)])
    plsc.subcore_barrier()

    wid = sid * _NC + cid
    pltpu.sync_copy(dst_hbm.at[wid], dst_v)

    def _body(j, _):
        pltpu.sync_copy(ones_v, acc.at[dst_v.at[j]], add=True)
        return 0
    lax.fori_loop(0, _C, _body, 0)
    plsc.subcore_barrier()

    pltpu.sync_copy(acc.at[pl.ds(sid * _DEG_PER_TILE, _DEG_PER_TILE)],
                    zeros_v.at[pl.ds(0, _DEG_PER_TILE)])
    pltpu.sync_copy(zeros_v.at[pl.ds(0, _DEG_PER_TILE)],
                    deg_hbm.at[pl.ds(cid * _DEGP + sid * _DEG_PER_TILE,
                                     _DEG_PER_TILE)])


def _sc_neighbor_sum_impl(hs_hbm, src_hbm, dst_hbm, out_hbm,
                          src_v, dst_v, buf, acc, sem):
    cid = lax.axis_index("c")
    sid = lax.axis_index("s")

    def _fill(i, _):
        r = i // 8
        c = lax.rem(i, 8) * 16
        buf[r, pl.ds(c, 16)] = jnp.zeros((16,), jnp.float32)
        return 0
    lax.fori_loop(0, _CH * 8, _fill, 0)

    zb = sid * _OUT_PER_TILE
    for t in range(_OUT_PER_TILE // _CH):
        pltpu.sync_copy(buf, acc.at[pl.ds(zb + t * _CH, _CH)])
    plsc.subcore_barrier()

    wid = sid * _NC + cid
    pltpu.sync_copy(src_hbm.at[wid], src_v)
    pltpu.sync_copy(dst_hbm.at[wid], dst_v)

    def _body(j, _):
        pltpu.async_copy(hs_hbm.at[src_v.at[j]], buf, sem).wait()
        pltpu.sync_copy(buf, acc.at[dst_v.at[j]], add=True)
        return 0
    lax.fori_loop(0, _C, _body, 0)
    plsc.subcore_barrier()

    ob = sid * _OUT_PER_TILE
    for t in range(_OUT_PER_TILE // _CH):
        pltpu.sync_copy(acc.at[pl.ds(ob + t * _CH, _CH)], buf)
        pltpu.sync_copy(buf, out_hbm.at[cid, pl.ds(ob + t * _CH, _CH)])


_BLK = 1000
_GRID = _N // _BLK


def _tc_first_body(d0_ref, d1_ref, x_ref, w_ref, hs_ref, dinv_ref):
    deg = d0_ref[...] + d1_ref[...] + 1.0
    dinv = lax.rsqrt(jnp.maximum(deg, 1.0))
    h = jnp.dot(x_ref[...], w_ref[...], preferred_element_type=jnp.float32)
    hs_ref[...] = dinv * h
    dinv_ref[...] = dinv


def _tc_mid_body(p0_ref, p1_ref, hs_ref, dinv_ref, b_ref, w_ref, out_ref):
    dinv = dinv_ref[...]
    agg = p0_ref[...] + p1_ref[...] + hs_ref[...]
    t = jnp.maximum(dinv * agg + b_ref[...], 0.0)
    out_ref[...] = dinv * jnp.dot(t, w_ref[...], preferred_element_type=jnp.float32)


def _tc_final_body(p0_ref, p1_ref, hs_ref, dinv_ref, b_ref, out_ref):
    agg = p0_ref[...] + p1_ref[...] + hs_ref[...]
    out_ref[...] = dinv_ref[...] * agg + b_ref[...]


_row_spec = pl.BlockSpec((_BLK, _D), lambda i: (i, 0))
_col_spec = pl.BlockSpec((_BLK, 1), lambda i: (i, 0))
_w_spec = pl.BlockSpec((_D, _D), lambda i: (0, 0))
_b_spec = pl.BlockSpec((1, _D), lambda i: (0, 0))

_tc_first = pl.pallas_call(
    _tc_first_body,
    grid=(_GRID,),
    in_specs=[_col_spec, _col_spec, _row_spec, _w_spec],
    out_specs=[_row_spec, _col_spec],
    out_shape=[jax.ShapeDtypeStruct((_N, _D), jnp.float32),
               jax.ShapeDtypeStruct((_N, 1), jnp.float32)],
)

_tc_mid = pl.pallas_call(
    _tc_mid_body,
    grid=(_GRID,),
    in_specs=[_row_spec, _row_spec, _row_spec, _col_spec, _b_spec, _w_spec],
    out_specs=_row_spec,
    out_shape=jax.ShapeDtypeStruct((_N, _D), jnp.float32),
)

_tc_final = pl.pallas_call(
    _tc_final_body,
    grid=(_GRID,),
    in_specs=[_row_spec, _row_spec, _row_spec, _col_spec, _b_spec],
    out_specs=_row_spec,
    out_shape=jax.ShapeDtypeStruct((_N, _D), jnp.float32),
)


@functools.lru_cache(maxsize=1)
def _sc_kernels():
    mesh = plsc.VectorSubcoreMesh(
        core_axis_name="c", subcore_axis_name="s",
        num_cores=_NC, num_subcores=_NS)
    sc_degree = pl.kernel(
        _sc_degree_impl,
        out_type=jax.ShapeDtypeStruct((_NC * _DEGP,), jnp.float32),
        mesh=mesh,
        scratch_types=[
            pltpu.VMEM((_C, _CH), jnp.int32),
            pltpu.VMEM((640,), jnp.float32),
            pltpu.VMEM((_CH,), jnp.float32),
            pltpu.VMEM_SHARED((_DEGP,), jnp.float32),
        ],
    )
    sc_neighbor_sum = pl.kernel(
        _sc_neighbor_sum_impl,
        out_type=jax.ShapeDtypeStruct((_NC, _ACC_ROWS, _D), jnp.float32),
        mesh=mesh,
        scratch_types=[
            pltpu.VMEM((_C, _CH), jnp.int32),
            pltpu.VMEM((_C, _CH), jnp.int32),
            pltpu.VMEM((_CH, _D), jnp.float32),
            pltpu.VMEM_SHARED((_ACC_ROWS, _D), jnp.float32),
            pltpu.SemaphoreType.DMA,
        ],
    )
    return sc_degree, sc_neighbor_sum


def kernel(x, edge_index, W1, b1, W2, b2, W3, b3):
    src = edge_index[0]
    dst = edge_index[1]
    pad = _EP - _E
    srcp = jnp.concatenate(
        [src, jnp.zeros((pad,), jnp.int32)]).reshape(_NW, _C, _CH)
    dstp = jnp.concatenate(
        [dst, jnp.full((pad,), _DUMMY, jnp.int32)]).reshape(_NW, _C, _CH)

    _sc_degree, _sc_neighbor_sum = _sc_kernels()
    deg2 = _sc_degree(dstp).reshape(_NC, _DEGP)
    d0 = deg2[0, :_N, None]
    d1 = deg2[1, :_N, None]

    b1r = b1.reshape(1, _D)
    b2r = b2.reshape(1, _D)
    b3r = b3.reshape(1, _D)

    hs1, dinv = _tc_first(d0, d1, x, W1)
    parts = _sc_neighbor_sum(hs1, srcp, dstp)
    hs2 = _tc_mid(parts[0, :_N], parts[1, :_N], hs1, dinv, b1r, W2)
    parts = _sc_neighbor_sum(hs2, srcp, dstp)
    hs3 = _tc_mid(parts[0, :_N], parts[1, :_N], hs2, dinv, b2r, W3)
    parts = _sc_neighbor_sum(hs3, srcp, dstp)
    return _tc_final(parts[0, :_N], parts[1, :_N], hs3, dinv, b3r)


# speedup vs baseline: 1.6003x; 1.5223x over previous
import functools

import jax
import jax.numpy as jnp
from jax import lax
from jax.experimental import pallas as pl
from jax.experimental.pallas import tpu as pltpu
from jax.experimental.pallas import tpu_sc as plsc

_N = 10000
_E = 320000
_D = 128

_NC = 2
_NS = 16
_NW = _NC * _NS
_CH = 128
_C = 79
_EP = _NW * _C * _CH
_DUMMY = _N

_ACC_ROWS = 10240
_OUT_PER_TILE = _ACC_ROWS // _NS

_DEG_PER_TILE = 632
_DEGP = _NS * _DEG_PER_TILE


def _sc_degree_impl(dst_hbm, deg_hbm, dst_v, zeros_v, ones_v, acc):
    cid = lax.axis_index("c")
    sid = lax.axis_index("s")

    def _fill(i, _):
        zeros_v[pl.ds(i * 16, 16)] = jnp.zeros((16,), jnp.float32)
        return 0
    lax.fori_loop(0, 640 // 16, _fill, 0)

    def _fill1(i, _):
        ones_v[pl.ds(i * 16, 16)] = jnp.ones((16,), jnp.float32)
        return 0
    lax.fori_loop(0, _CH // 16, _fill1, 0)

    pltpu.sync_copy(zeros_v.at[pl.ds(0, _DEG_PER_TILE)],
                    acc.at[pl.ds(sid * _DEG_PER_TILE, _DEG_PER_TILE)])
    plsc.subcore_barrier()

    wid = sid * _NC + cid
    pltpu.sync_copy(dst_hbm.at[wid], dst_v)

    def _body(j, _):
        pltpu.sync_copy(ones_v, acc.at[dst_v.at[j]], add=True)
        return 0
    lax.fori_loop(0, _C, _body, 0)
    plsc.subcore_barrier()

    pltpu.sync_copy(acc.at[pl.ds(sid * _DEG_PER_TILE, _DEG_PER_TILE)],
                    zeros_v.at[pl.ds(0, _DEG_PER_TILE)])
    pltpu.sync_copy(zeros_v.at[pl.ds(0, _DEG_PER_TILE)],
                    deg_hbm.at[pl.ds(cid * _DEGP + sid * _DEG_PER_TILE,
                                     _DEG_PER_TILE)])


def _sc_neighbor_sum_impl(hs_hbm, src_hbm, dst_hbm, out_hbm,
                          src_v, dst_v, buf, acc, sem):
    cid = lax.axis_index("c")
    sid = lax.axis_index("s")

    def _fill(i, _):
        r = i // 8
        c = lax.rem(i, 8) * 16
        buf[r, pl.ds(c, 16)] = jnp.zeros((16,), jnp.float32)
        return 0
    lax.fori_loop(0, _CH * 8, _fill, 0)

    zb = sid * _OUT_PER_TILE
    for t in range(_OUT_PER_TILE // _CH):
        pltpu.sync_copy(buf, acc.at[pl.ds(zb + t * _CH, _CH)])
    plsc.subcore_barrier()

    wid = sid * _NC + cid
    pltpu.sync_copy(src_hbm.at[wid], src_v)
    pltpu.sync_copy(dst_hbm.at[wid], dst_v)

    def _body(j, _):
        pltpu.async_copy(hs_hbm.at[src_v.at[j]], buf, sem).wait()
        pltpu.sync_copy(buf, acc.at[dst_v.at[j]], add=True)
        return 0
    lax.fori_loop(0, _C, _body, 0)
    plsc.subcore_barrier()

    ob = sid * _OUT_PER_TILE
    for t in range(_OUT_PER_TILE // _CH):
        pltpu.sync_copy(acc.at[pl.ds(ob + t * _CH, _CH)], buf)
        pltpu.sync_copy(buf, out_hbm.at[cid, pl.ds(ob + t * _CH, _CH)])


_BLK = 1000
_GRID = _N // _BLK


def _tc_first_body(d0_ref, d1_ref, x_ref, w_ref, hs_ref, dinv_ref):
    deg = d0_ref[...] + d1_ref[...] + 1.0
    dinv = lax.rsqrt(jnp.maximum(deg, 1.0))
    h = jnp.dot(x_ref[...], w_ref[...], preferred_element_type=jnp.float32)
    hs_ref[...] = dinv * h
    dinv_ref[...] = dinv


def _tc_mid_body(p0_ref, p1_ref, hs_ref, dinv_ref, b_ref, w_ref, out_ref):
    dinv = dinv_ref[...]
    agg = p0_ref[...] + p1_ref[...] + hs_ref[...]
    t = jnp.maximum(dinv * agg + b_ref[...], 0.0)
    out_ref[...] = dinv * jnp.dot(t, w_ref[...], preferred_element_type=jnp.float32)


def _tc_final_body(p0_ref, p1_ref, hs_ref, dinv_ref, b_ref, out_ref):
    agg = p0_ref[...] + p1_ref[...] + hs_ref[...]
    out_ref[...] = dinv_ref[...] * agg + b_ref[...]


_row_spec = pl.BlockSpec((_BLK, _D), lambda i: (i, 0))
_col_spec = pl.BlockSpec((_BLK, 1), lambda i: (i, 0))
_w_spec = pl.BlockSpec((_D, _D), lambda i: (0, 0))
_b_spec = pl.BlockSpec((1, _D), lambda i: (0, 0))

_tc_first = pl.pallas_call(
    _tc_first_body,
    grid=(_GRID,),
    in_specs=[_col_spec, _col_spec, _row_spec, _w_spec],
    out_specs=[_row_spec, _col_spec],
    out_shape=[jax.ShapeDtypeStruct((_N, _D), jnp.float32),
               jax.ShapeDtypeStruct((_N, 1), jnp.float32)],
)

_tc_mid = pl.pallas_call(
    _tc_mid_body,
    grid=(_GRID,),
    in_specs=[_row_spec, _row_spec, _row_spec, _col_spec, _b_spec, _w_spec],
    out_specs=_row_spec,
    out_shape=jax.ShapeDtypeStruct((_N, _D), jnp.float32),
)

_tc_final = pl.pallas_call(
    _tc_final_body,
    grid=(_GRID,),
    in_specs=[_row_spec, _row_spec, _row_spec, _col_spec, _b_spec],
    out_specs=_row_spec,
    out_shape=jax.ShapeDtypeStruct((_N, _D), jnp.float32),
)


@functools.lru_cache(maxsize=1)
def _sc_kernels():
    mesh = plsc.VectorSubcoreMesh(
        core_axis_name="c", subcore_axis_name="s",
        num_cores=_NC, num_subcores=_NS)
    sc_degree = pl.kernel(
        _sc_degree_impl,
        out_type=jax.ShapeDtypeStruct((_NC * _DEGP,), jnp.float32),
        mesh=mesh,
        scratch_types=[
            pltpu.VMEM((_C, _CH), jnp.int32),
            pltpu.VMEM((640,), jnp.float32),
            pltpu.VMEM((_CH,), jnp.float32),
            pltpu.VMEM_SHARED((_DEGP,), jnp.float32),
        ],
    )
    sc_neighbor_sum = pl.kernel(
        _sc_neighbor_sum_impl,
        out_type=jax.ShapeDtypeStruct((_NC, _ACC_ROWS, _D), jnp.float32),
        mesh=mesh,
        scratch_types=[
            pltpu.VMEM((_C, _CH), jnp.int32),
            pltpu.VMEM((_C, _CH), jnp.int32),
            pltpu.VMEM((_CH, _D), jnp.float32),
            pltpu.VMEM_SHARED((_ACC_ROWS, _D), jnp.float32),
            pltpu.SemaphoreType.DMA,
        ],
    )
    return sc_degree, sc_neighbor_sum


def kernel(x, edge_index, W1, b1, W2, b2, W3, b3):
    src = edge_index[0]
    dst = edge_index[1]
    pad = _EP - _E
    srcp = jnp.concatenate(
        [src, jnp.zeros((pad,), jnp.int32)]).reshape(_NW, _C, _CH)
    dstp = jnp.concatenate(
        [dst, jnp.full((pad,), _DUMMY, jnp.int32)]).reshape(_NW, _C, _CH)

    _sc_degree, _sc_neighbor_sum = _sc_kernels()
    deg2 = _sc_degree(dstp).reshape(_NC, _DEGP)
    d0 = deg2[0, :_N, None]
    d1 = deg2[1, :_N, None]

    b1r = b1.reshape(1, _D)
    b2r = b2.reshape(1, _D)
    b3r = b3.reshape(1, _D)

    hs1, dinv = _tc_first(d0, d1, x, W1)
    parts = _sc_neighbor_sum(hs1, srcp, dstp)
    hs2 = _tc_mid(parts[0, :_N], parts[1, :_N], hs1, dinv, b1r, W2)
    parts = _sc_neighbor_sum(hs2, srcp, dstp)
    hs3 = _tc_mid(parts[0, :_N], parts[1, :_N], hs2, dinv, b2r, W3)
    parts = _sc_neighbor_sum(hs3, srcp, dstp)
    return _tc_final(parts[0, :_N], parts[1, :_N], hs3, dinv, b3r)
